# Initial kernel scaffold; baseline (speedup 1.0000x reference)
import jax
import jax.numpy as jnp
from jax.experimental import pallas as pl


def kernel(x, boundaries, W_orig, b_orig, W_new, perm, saved_masks):
    raise NotImplementedError("write your pallas kernel here")


# speedup vs baseline: 2.0342x; 2.0342x over previous
import functools

import jax
import jax.numpy as jnp
from jax.experimental import pallas as pl
from jax.experimental.pallas import tpu as pltpu

B = 4
S = 2048
D = 1024
TOP_K = 512
M_SAVED = 32
IRR_THRESHOLD = 0.5
S_TILE = 512


def _mask_kernel(bound_ref, x_ref, perm_ref, saved_t_ref, mask_out_ref,
                 agg_ref):
    b = pl.program_id(0)
    nb = pl.num_programs(0)

    bound = jnp.clip(bound_ref[b], 0, S - 1)
    pos = jax.lax.broadcasted_iota(jnp.int32, (S, D), 0)
    pmask = (pos <= bound).astype(jnp.float32)
    denom = (bound + 1).astype(jnp.float32)
    agg_row = jnp.sum(x_ref[0] * pmask, axis=0, keepdims=True) / denom
    agg_ref[pl.ds(b, 1), :] = agg_row

    @pl.when(b == nb - 1)
    def _():
        ident = (jax.lax.broadcasted_iota(jnp.int32, (D, D), 0) ==
                 jax.lax.broadcasted_iota(jnp.int32, (D, D), 1)
                 ).astype(jnp.float32)
        perm_eq = (perm_ref[...] ==
                   jax.lax.broadcasted_iota(jnp.int32, (D, D), 1)
                   ).astype(jnp.float32)
        saved_perm_t = jax.lax.dot_general(
            perm_eq, saved_t_ref[...],
            (((1,), (0,)), ((), ())),
            preferred_element_type=jnp.float32)
        for bb in range(B):
            abs_row = jnp.abs(agg_ref[pl.ds(bb, 1), :])
            abs_col = jax.lax.dot_general(
                ident, abs_row, (((1,), (1,)), ((), ())),
                precision=jax.lax.Precision.HIGHEST,
                preferred_element_type=jnp.float32)
            a_row = jax.lax.broadcast_in_dim(abs_row, (D, D), (0, 1))
            a_col = jax.lax.broadcast_in_dim(abs_col, (D, D), (0, 1))
            gt = (a_col > a_row).astype(jnp.float32)
            tie = ((a_col == a_row) &
                   (jax.lax.broadcasted_iota(jnp.int32, (D, D), 0) <
                    jax.lax.broadcasted_iota(jnp.int32, (D, D), 1))
                   ).astype(jnp.float32)
            rank = jnp.sum(gt + tie, axis=0, keepdims=True)
            selected = (rank < float(TOP_K)).astype(jnp.float32)
            counts = jax.lax.dot_general(
                selected, saved_perm_t, (((1,), (0,)), ((), ())),
                preferred_element_type=jnp.float32)
            best_count = jnp.max(counts)
            relevant = best_count >= float(IRR_THRESHOLD) * float(TOP_K)
            m_iota = jax.lax.broadcasted_iota(
                jnp.int32, (1, M_SAVED), 1).astype(jnp.float32)
            key = counts * float(M_SAVED) + (float(M_SAVED - 1) - m_iota)
            onehot = (key == jnp.max(key)).astype(jnp.float32)
            best_mask = jax.lax.dot_general(
                onehot, saved_t_ref[...], (((1,), (1,)), ((), ())),
                preferred_element_type=jnp.float32)
            mask_out_ref[pl.ds(bb, 1), :] = jnp.where(
                relevant, best_mask, jnp.zeros_like(best_mask))


def _matmul_kernel(x_ref, w_orig_ref, w_new_ref, mask_ref, bias_ref,
                   out_ref, wc_ref):
    s = pl.program_id(1)

    @pl.when(s == 0)
    def _():
        wc_ref[...] = (w_orig_ref[...] +
                       mask_ref[0] * w_new_ref[...]).astype(jnp.bfloat16)

    x_bf = x_ref[0].astype(jnp.bfloat16)
    acc = jax.lax.dot_general(
        x_bf, wc_ref[...], (((1,), (1,)), ((), ())),
        preferred_element_type=jnp.float32)
    out_ref[0] = acc + bias_ref[...]


@functools.partial(jax.jit, static_argnames=())
def kernel(x, boundaries, W_orig, b_orig, W_new, perm, saved_masks):
    boundaries = boundaries.astype(jnp.int32)
    saved_t = saved_masks.T.astype(jnp.float32)
    perm_col = perm.reshape(D, 1)
    bias = b_orig.reshape(1, D)

    masks = pl.pallas_call(
        _mask_kernel,
        grid_spec=pltpu.PrefetchScalarGridSpec(
            num_scalar_prefetch=1,
            grid=(B,),
            in_specs=[
                pl.BlockSpec((1, S, D), lambda b, *_: (b, 0, 0)),
                pl.BlockSpec((D, 1), lambda b, *_: (0, 0)),
                pl.BlockSpec((D, M_SAVED), lambda b, *_: (0, 0)),
            ],
            out_specs=pl.BlockSpec((B, D), lambda b, *_: (0, 0)),
            scratch_shapes=[pltpu.VMEM((B, D), jnp.float32)],
        ),
        out_shape=jax.ShapeDtypeStruct((B, D), jnp.float32),
        compiler_params=pltpu.CompilerParams(
            dimension_semantics=("arbitrary",)),
    )(boundaries, x, perm_col, saved_t)
    masks3 = masks.reshape(B, 1, D)

    out = pl.pallas_call(
        _matmul_kernel,
        grid=(B, S // S_TILE),
        in_specs=[
            pl.BlockSpec((1, S_TILE, D), lambda b, s: (b, s, 0)),
            pl.BlockSpec((D, D), lambda b, s: (0, 0)),
            pl.BlockSpec((D, D), lambda b, s: (0, 0)),
            pl.BlockSpec((1, 1, D), lambda b, s: (b, 0, 0)),
            pl.BlockSpec((1, D), lambda b, s: (0, 0)),
        ],
        out_specs=pl.BlockSpec((1, S_TILE, D), lambda b, s: (b, s, 0)),
        out_shape=jax.ShapeDtypeStruct((B, S, D), jnp.float32),
        scratch_shapes=[pltpu.VMEM((D, D), jnp.bfloat16)],
        compiler_params=pltpu.CompilerParams(
            dimension_semantics=("arbitrary", "arbitrary")),
    )(x, W_orig, W_new, masks3, bias)
    return out
